# Initial kernel scaffold; baseline (speedup 1.0000x reference)
import jax
import jax.numpy as jnp
from jax.experimental import pallas as pl


def kernel(x, edge_index, edge_attr, action, W1, b1, W2, b2, Wl1, bl1, Wg, bg):
    raise NotImplementedError("write your pallas kernel here")


# speedup vs baseline: 3.0169x; 3.0169x over previous
import functools

import jax
import jax.numpy as jnp
from jax import lax
from jax.experimental import pallas as pl
from jax.experimental.pallas import tpu as pltpu
from jax.experimental.pallas import tpu_sc as plsc

_NNODES = 50
_NODE = 16
_EATTR = 4
_HID = 32
_NFACT = 3
_B = 2000
_N = _B * _NNODES
_E = 1600000
_NGRP = _E // 128
_GPC = 20
_ECH = _GPC * 128
_NCH = _NGRP // _GPC
_NC = 2
_NS = 16
_HALF = _N // _NC
_ZR = 3128
_ACCR = _NS * _ZR
_DUMMY = _HALF + 8

_MESH = dict(core_axis_name="c", subcore_axis_name="s")


def _sc_gather(x, src, dst):
    def body(x_hbm, src_hbm, dst_hbm, gs_hbm, gd_hbm,
             idx_s, idx_d, buf_s, buf_d, sem_s, sem_d):
        c = lax.axis_index("c")
        s = lax.axis_index("s")
        w = c * _NS + s
        nq = jnp.where(w < _NCH % 32, _NCH // 32 + 1, _NCH // 32)

        def chunk(k, carry):
            q = w + k * 32
            e0 = q * _ECH
            pltpu.sync_copy(src_hbm.at[pl.ds(e0, _ECH)], idx_s)
            pltpu.sync_copy(dst_hbm.at[pl.ds(e0, _ECH)], idx_d)
            cps = [pltpu.async_copy(x_hbm.at[idx_s.at[pl.ds(j * 128, 128)]],
                                    buf_s.at[pl.ds(j * 128, 128)], sem_s)
                   for j in range(_GPC)]
            cpd = [pltpu.async_copy(x_hbm.at[idx_d.at[pl.ds(j * 128, 128)]],
                                    buf_d.at[pl.ds(j * 128, 128)], sem_d)
                   for j in range(_GPC)]
            for cp in cps:
                cp.wait()
            for cp in cpd:
                cp.wait()
            pltpu.sync_copy(buf_s, gs_hbm.at[pl.ds(e0, _ECH)])
            pltpu.sync_copy(buf_d, gd_hbm.at[pl.ds(e0, _ECH)])
            return carry

        lax.fori_loop(0, nq, chunk, 0)

    f = pl.kernel(
        body,
        mesh=plsc.VectorSubcoreMesh(**_MESH),
        compiler_params=pltpu.CompilerParams(use_tc_tiling_on_sc=False),
        out_type=[jax.ShapeDtypeStruct((_E, _NODE), jnp.float32),
                  jax.ShapeDtypeStruct((_E, _NODE), jnp.float32)],
        scratch_types=[
            pltpu.VMEM((_ECH,), jnp.int32),
            pltpu.VMEM((_ECH,), jnp.int32),
            pltpu.VMEM((_ECH, _NODE), jnp.float32),
            pltpu.VMEM((_ECH, _NODE), jnp.float32),
            pltpu.SemaphoreType.DMA,
            pltpu.SemaphoreType.DMA,
        ],
    )
    return f(x, src, dst)


_EB = 8000


def _edge_mlp_body(gs, gd, ea, w1a, w1b, w1c, b1, w2, b2, out):
    h = (jnp.dot(gs[...], w1a[...], preferred_element_type=jnp.float32)
         + jnp.dot(gd[...], w1b[...], preferred_element_type=jnp.float32)
         + jnp.dot(ea[...], w1c[...], preferred_element_type=jnp.float32)
         + b1[...])
    h = jnp.maximum(h, 0.0)
    out[...] = jnp.dot(h, w2[...], preferred_element_type=jnp.float32) + b2[...]


def _edge_mlp(gs, gd, ea, w1a, w1b, w1c, b1, w2, b2):
    nsteps = _E // _EB
    full = lambda shape: pl.BlockSpec(shape, lambda i: (0, 0))
    return pl.pallas_call(
        _edge_mlp_body,
        grid=(nsteps,),
        in_specs=[
            pl.BlockSpec((_EB, _NODE), lambda i: (i, 0)),
            pl.BlockSpec((_EB, _NODE), lambda i: (i, 0)),
            pl.BlockSpec((_EB, _EATTR), lambda i: (i, 0)),
            full((_NODE, _HID)),
            full((_NODE, _HID)),
            full((_EATTR, _HID)),
            full((1, _HID)),
            full((_HID, _HID)),
            full((1, _HID)),
        ],
        out_specs=pl.BlockSpec((_EB, _HID), lambda i: (i, 0)),
        out_shape=jax.ShapeDtypeStruct((_E, _HID), jnp.float32),
    )(gs, gd, ea, w1a, w1b, w1c, b1, w2, b2)


_GPS = 5
_ECS = _GPS * 128
_NCS = _NGRP // _GPS


def _sc_scatter(src, msg):
    def body(src_hbm, msg_hbm, xpp_hbm, src_v, idx_v, msg_v, acc, sem):
        c = lax.axis_index("c")
        s = lax.axis_index("s")
        node_base = c * _HALF

        def zrow(r, carry):
            z = jnp.zeros((16,), jnp.float32)
            msg_v[r, pl.ds(0, 16)] = z
            msg_v[r, pl.ds(16, 16)] = z
            return carry
        lax.fori_loop(0, _ECS, zrow, 0)
        a0 = s * _ZR
        for t in range(4):
            pltpu.sync_copy(msg_v, acc.at[pl.ds(a0 + t * _ECS, _ECS)])
        pltpu.sync_copy(msg_v.at[pl.ds(0, _ZR - 4 * _ECS)],
                        acc.at[pl.ds(a0 + 4 * _ECS, _ZR - 4 * _ECS)])
        plsc.subcore_barrier()

        nq = jnp.where(s < _NCS % _NS, _NCS // _NS + 1, _NCS // _NS)

        def chunk(k, carry):
            q = s + k * _NS
            e0 = q * _ECS
            pltpu.sync_copy(src_hbm.at[pl.ds(e0, _ECS)], src_v)
            pltpu.sync_copy(msg_hbm.at[pl.ds(e0, _ECS)], msg_v)

            def cidx(i, carry2):
                j = i // 8
                o = (i % 8) * 16
                sv = src_v[pl.ds(i * 16, 16)]
                loc = sv - node_base
                ok = (loc >= 0) & (loc < _HALF)
                idx_v[j, pl.ds(o, 16)] = jnp.where(ok, loc, _DUMMY)
                return carry2
            lax.fori_loop(0, _GPS * 8, cidx, 0)

            for j in range(_GPS):
                pltpu.sync_copy(msg_v.at[pl.ds(j * 128, 128)],
                                acc.at[idx_v.at[j]], add=True)
            return carry

        lax.fori_loop(0, nq, chunk, 0)
        plsc.subcore_barrier()

        last = _HALF - (_NS - 1) * _ZR

        @pl.when(s < _NS - 1)
        def _():
            pltpu.sync_copy(acc.at[pl.ds(a0, _ZR)],
                            xpp_hbm.at[pl.ds(node_base + a0, _ZR)])

        @pl.when(s == _NS - 1)
        def _():
            b = (_NS - 1) * _ZR
            pltpu.sync_copy(acc.at[pl.ds(b, last)],
                            xpp_hbm.at[pl.ds(node_base + b, last)])

    f = pl.kernel(
        body,
        mesh=plsc.VectorSubcoreMesh(**_MESH),
        compiler_params=pltpu.CompilerParams(use_tc_tiling_on_sc=False),
        out_type=jax.ShapeDtypeStruct((_N, _HID), jnp.float32),
        scratch_types=[
            pltpu.VMEM((_ECS,), jnp.int32),
            pltpu.VMEM((_GPS, 128), jnp.int32),
            pltpu.VMEM((_ECS, _HID), jnp.float32),
            pltpu.VMEM_SHARED((_ACCR, _HID), jnp.float32),
            pltpu.SemaphoreType.DMA,
        ],
    )
    return f(src, msg)


_GB = 40
_RB = _GB * _NNODES


def _head_body(x, xpp, xr, xppr, am, ax, wl1, bl1, wg, bgs, out):
    wa1 = wl1[0:16, :]
    wa2 = wl1[16:48, :]
    wb1 = wl1[48:64, :]
    wb2 = wl1[64:96, :]
    wact = wl1[96:97, :]
    u = (jnp.dot(x[...], wa1, preferred_element_type=jnp.float32)
         + jnp.dot(xpp[...], wa2, preferred_element_type=jnp.float32))
    vr = (jnp.dot(xr[...], wb1, preferred_element_type=jnp.float32)
          + jnp.dot(xppr[...], wb2, preferred_element_type=jnp.float32))
    v = (jnp.dot(x[...], wb1, preferred_element_type=jnp.float32)
         + jnp.dot(xpp[...], wb2, preferred_element_type=jnp.float32))
    h1 = jnp.maximum(u + vr + am[...] * wact + bl1[...], 0.0)
    q = jnp.dot(h1, wg[...], preferred_element_type=jnp.float32)
    kmod = lax.broadcasted_iota(jnp.int32, (_RB, 1), 0) % _NNODES
    h2 = jnp.maximum(u + v + ax[...] * wact + bl1[...], 0.0)
    q2 = jnp.dot(h2, wg[...], preferred_element_type=jnp.float32)
    q = q + jnp.where(kmod >= _NNODES - _NFACT, q2, 0.0)
    gsel = (lax.broadcasted_iota(jnp.int32, (_GB, _RB), 0)
            == lax.broadcasted_iota(jnp.int32, (_GB, _RB), 1) // _NNODES)
    out[...] = (jnp.dot(gsel.astype(jnp.float32), q,
                        preferred_element_type=jnp.float32)
                + (_NNODES + _NFACT) * bgs[...])


def _head(x, xpp, xr, xppr, am, ax, wl1, bl1, wg, bgs):
    nsteps = _B // _GB
    full = lambda shape: pl.BlockSpec(shape, lambda i: (0, 0))
    return pl.pallas_call(
        _head_body,
        grid=(nsteps,),
        in_specs=[
            pl.BlockSpec((_RB, _NODE), lambda i: (i, 0)),
            pl.BlockSpec((_RB, _HID), lambda i: (i, 0)),
            pl.BlockSpec((_RB, _NODE), lambda i: (i, 0)),
            pl.BlockSpec((_RB, _HID), lambda i: (i, 0)),
            pl.BlockSpec((_RB, 1), lambda i: (i, 0)),
            pl.BlockSpec((_RB, 1), lambda i: (i, 0)),
            full((2 * (_NODE + _HID) + 1, _HID)),
            full((1, _HID)),
            full((_HID, 1)),
            full((1, 1)),
        ],
        out_specs=pl.BlockSpec((_GB, 1), lambda i: (i, 0)),
        out_shape=jax.ShapeDtypeStruct((_B, 1), jnp.float32),
    )(x, xpp, xr, xppr, am, ax, wl1, bl1, wg, bgs)


def kernel(x, edge_index, edge_attr, action, W1, b1, W2, b2, Wl1, bl1, Wg, bg):
    src = edge_index[0]
    dst = edge_index[1]

    gs, gd = _sc_gather(x, src, dst)
    msg = _edge_mlp(gs, gd, edge_attr,
                    W1[:_NODE], W1[_NODE:2 * _NODE], W1[2 * _NODE:],
                    b1.reshape(1, _HID), W2, b2.reshape(1, _HID))
    xpp = _sc_scatter(src, msg)

    xr = jnp.roll(x.reshape(_B, _NNODES, _NODE), -1, axis=1).reshape(_N, _NODE)
    xppr = jnp.roll(xpp.reshape(_B, _NNODES, _HID), -1, axis=1).reshape(_N, _HID)
    am = action[:, :_NNODES].reshape(_N, 1)
    ax = jnp.concatenate(
        [jnp.zeros((_B, _NNODES - _NFACT), jnp.float32), action[:, _NNODES:]],
        axis=1).reshape(_N, 1)

    out = _head(x, xpp, xr, xppr, am, ax,
                Wl1, bl1.reshape(1, _HID), Wg, bg.reshape(1, 1))
    return out.reshape(_B)


# speedup vs baseline: 3.4579x; 1.1462x over previous
import functools

import jax
import jax.numpy as jnp
from jax import lax
from jax.experimental import pallas as pl
from jax.experimental.pallas import tpu as pltpu
from jax.experimental.pallas import tpu_sc as plsc

_NNODES = 50
_NODE = 16
_EATTR = 4
_HID = 32
_NFACT = 3
_B = 2000
_N = _B * _NNODES
_E = 1600000
_NGRP = _E // 128
_GPC = 20
_ECH = _GPC * 128
_NCH = _NGRP // _GPC
_NC = 2
_NS = 16
_HALF = _N // _NC
_ZR = 3128
_ACCR = _NS * _ZR
_DUMMY = _HALF + 8

_MESH = dict(core_axis_name="c", subcore_axis_name="s")


def _sc_gather(x, src, dst):
    def body(x_hbm, src_hbm, dst_hbm, gs_hbm, gd_hbm,
             idx_s, idx_d, buf_s, buf_d, sem_s, sem_d):
        c = lax.axis_index("c")
        s = lax.axis_index("s")
        w = c * _NS + s
        nq = jnp.where(w < _NCH % 32, _NCH // 32 + 1, _NCH // 32)

        def chunk(k, carry):
            q = w + k * 32
            e0 = q * _ECH
            pltpu.sync_copy(src_hbm.at[pl.ds(e0, _ECH)], idx_s)
            pltpu.sync_copy(dst_hbm.at[pl.ds(e0, _ECH)], idx_d)
            cps = [pltpu.async_copy(x_hbm.at[idx_s.at[pl.ds(j * 128, 128)]],
                                    buf_s.at[pl.ds(j * 128, 128)], sem_s)
                   for j in range(_GPC)]
            cpd = [pltpu.async_copy(x_hbm.at[idx_d.at[pl.ds(j * 128, 128)]],
                                    buf_d.at[pl.ds(j * 128, 128)], sem_d)
                   for j in range(_GPC)]
            for cp in cps:
                cp.wait()
            for cp in cpd:
                cp.wait()
            pltpu.sync_copy(buf_s, gs_hbm.at[pl.ds(e0, _ECH)])
            pltpu.sync_copy(buf_d, gd_hbm.at[pl.ds(e0, _ECH)])
            return carry

        lax.fori_loop(0, nq, chunk, 0)

    f = pl.kernel(
        body,
        mesh=plsc.VectorSubcoreMesh(**_MESH),
        compiler_params=pltpu.CompilerParams(use_tc_tiling_on_sc=False),
        out_type=[jax.ShapeDtypeStruct((_E, _NODE), jnp.float32),
                  jax.ShapeDtypeStruct((_E, _NODE), jnp.float32)],
        scratch_types=[
            pltpu.VMEM((_ECH,), jnp.int32),
            pltpu.VMEM((_ECH,), jnp.int32),
            pltpu.VMEM((_ECH, _NODE), jnp.float32),
            pltpu.VMEM((_ECH, _NODE), jnp.float32),
            pltpu.SemaphoreType.DMA,
            pltpu.SemaphoreType.DMA,
        ],
    )
    return f(x, src, dst)


_EB = 12800


def _edge_mlp_body(gs8, gd8, ea32, w1a, w1b, w1c, b1, w2, b2, out):
    r8 = _EB // 8
    r32 = _EB // 32
    r4 = _EB // 4
    t1 = jnp.dot(gs8[...], w1a[...], preferred_element_type=jnp.float32)
    t1 = t1.reshape(r8, 2, 128).reshape(r4, 128)
    t2 = jnp.dot(gd8[...], w1b[...], preferred_element_type=jnp.float32)
    t2 = t2.reshape(r8, 2, 128).reshape(r4, 128)
    t3 = jnp.dot(ea32[...], w1c[...], preferred_element_type=jnp.float32)
    t3 = t3.reshape(r32, 8, 128).reshape(r4, 128)
    h = jnp.maximum(t1 + t2 + t3 + b1[...], 0.0)
    out[...] = jnp.dot(h, w2[...], preferred_element_type=jnp.float32) + b2[...]


def _edge_mlp(gs8, gd8, ea32, w1a, w1b, w1c, b1, w2, b2):
    nsteps = _E // _EB
    full = lambda shape: pl.BlockSpec(shape, lambda i: (0, 0))
    return pl.pallas_call(
        _edge_mlp_body,
        grid=(nsteps,),
        in_specs=[
            pl.BlockSpec((_EB // 8, 128), lambda i: (i, 0)),
            pl.BlockSpec((_EB // 8, 128), lambda i: (i, 0)),
            pl.BlockSpec((_EB // 32, 128), lambda i: (i, 0)),
            full((128, 256)),
            full((128, 256)),
            full((128, 1024)),
            full((1, 128)),
            full((128, 128)),
            full((1, 128)),
        ],
        out_specs=pl.BlockSpec((_EB // 4, 128), lambda i: (i, 0)),
        out_shape=jax.ShapeDtypeStruct((_E // 4, 128), jnp.float32),
    )(gs8, gd8, ea32, w1a, w1b, w1c, b1, w2, b2)


_GPS = 5
_ECS = _GPS * 128
_NCS = _NGRP // _GPS


def _sc_scatter(src, msg):
    def body(src_hbm, msg_hbm, xpp_hbm, src_v, idx_v, msg_v, acc, sem):
        c = lax.axis_index("c")
        s = lax.axis_index("s")
        node_base = c * _HALF

        def zrow(r, carry):
            z = jnp.zeros((16,), jnp.float32)
            msg_v[r, pl.ds(0, 16)] = z
            msg_v[r, pl.ds(16, 16)] = z
            return carry
        lax.fori_loop(0, _ECS, zrow, 0)
        a0 = s * _ZR
        for t in range(4):
            pltpu.sync_copy(msg_v, acc.at[pl.ds(a0 + t * _ECS, _ECS)])
        pltpu.sync_copy(msg_v.at[pl.ds(0, _ZR - 4 * _ECS)],
                        acc.at[pl.ds(a0 + 4 * _ECS, _ZR - 4 * _ECS)])
        plsc.subcore_barrier()

        nq = jnp.where(s < _NCS % _NS, _NCS // _NS + 1, _NCS // _NS)

        def chunk(k, carry):
            q = s + k * _NS
            e0 = q * _ECS
            pltpu.sync_copy(src_hbm.at[pl.ds(e0, _ECS)], src_v)
            pltpu.sync_copy(msg_hbm.at[pl.ds(e0, _ECS)], msg_v)

            def cidx(i, carry2):
                j = i // 8
                o = (i % 8) * 16
                sv = src_v[pl.ds(i * 16, 16)]
                loc = sv - node_base
                ok = (loc >= 0) & (loc < _HALF)
                idx_v[j, pl.ds(o, 16)] = jnp.where(ok, loc, _DUMMY)
                return carry2
            lax.fori_loop(0, _GPS * 8, cidx, 0)

            for j in range(_GPS):
                pltpu.sync_copy(msg_v.at[pl.ds(j * 128, 128)],
                                acc.at[idx_v.at[j]], add=True)
            return carry

        lax.fori_loop(0, nq, chunk, 0)
        plsc.subcore_barrier()

        last = _HALF - (_NS - 1) * _ZR

        @pl.when(s < _NS - 1)
        def _():
            pltpu.sync_copy(acc.at[pl.ds(a0, _ZR)],
                            xpp_hbm.at[pl.ds(node_base + a0, _ZR)])

        @pl.when(s == _NS - 1)
        def _():
            b = (_NS - 1) * _ZR
            pltpu.sync_copy(acc.at[pl.ds(b, last)],
                            xpp_hbm.at[pl.ds(node_base + b, last)])

    f = pl.kernel(
        body,
        mesh=plsc.VectorSubcoreMesh(**_MESH),
        compiler_params=pltpu.CompilerParams(use_tc_tiling_on_sc=False),
        out_type=jax.ShapeDtypeStruct((_N, _HID), jnp.float32),
        scratch_types=[
            pltpu.VMEM((_ECS,), jnp.int32),
            pltpu.VMEM((_GPS, 128), jnp.int32),
            pltpu.VMEM((_ECS, _HID), jnp.float32),
            pltpu.VMEM_SHARED((_ACCR, _HID), jnp.float32),
            pltpu.SemaphoreType.DMA,
        ],
    )
    return f(src, msg)


_GB = 40
_RB = _GB * _NNODES


def _head_body(x, xpp, xr, xppr, am, ax, wl1, bl1, wg, bgs, out):
    wa1 = wl1[0:16, :]
    wa2 = wl1[16:48, :]
    wb1 = wl1[48:64, :]
    wb2 = wl1[64:96, :]
    wact = wl1[96:97, :]
    u = (jnp.dot(x[...], wa1, preferred_element_type=jnp.float32)
         + jnp.dot(xpp[...], wa2, preferred_element_type=jnp.float32))
    vr = (jnp.dot(xr[...], wb1, preferred_element_type=jnp.float32)
          + jnp.dot(xppr[...], wb2, preferred_element_type=jnp.float32))
    v = (jnp.dot(x[...], wb1, preferred_element_type=jnp.float32)
         + jnp.dot(xpp[...], wb2, preferred_element_type=jnp.float32))
    h1 = jnp.maximum(u + vr + am[...] * wact + bl1[...], 0.0)
    q = jnp.dot(h1, wg[...], preferred_element_type=jnp.float32)
    kmod = lax.broadcasted_iota(jnp.int32, (_RB, 1), 0) % _NNODES
    h2 = jnp.maximum(u + v + ax[...] * wact + bl1[...], 0.0)
    q2 = jnp.dot(h2, wg[...], preferred_element_type=jnp.float32)
    q = q + jnp.where(kmod >= _NNODES - _NFACT, q2, 0.0)
    gsel = (lax.broadcasted_iota(jnp.int32, (_GB, _RB), 0)
            == lax.broadcasted_iota(jnp.int32, (_GB, _RB), 1) // _NNODES)
    out[...] = (jnp.dot(gsel.astype(jnp.float32), q,
                        preferred_element_type=jnp.float32)
                + (_NNODES + _NFACT) * bgs[...])


def _head(x, xpp, xr, xppr, am, ax, wl1, bl1, wg, bgs):
    nsteps = _B // _GB
    full = lambda shape: pl.BlockSpec(shape, lambda i: (0, 0))
    return pl.pallas_call(
        _head_body,
        grid=(nsteps,),
        in_specs=[
            pl.BlockSpec((_RB, _NODE), lambda i: (i, 0)),
            pl.BlockSpec((_RB, _HID), lambda i: (i, 0)),
            pl.BlockSpec((_RB, _NODE), lambda i: (i, 0)),
            pl.BlockSpec((_RB, _HID), lambda i: (i, 0)),
            pl.BlockSpec((_RB, 1), lambda i: (i, 0)),
            pl.BlockSpec((_RB, 1), lambda i: (i, 0)),
            full((2 * (_NODE + _HID) + 1, _HID)),
            full((1, _HID)),
            full((_HID, 1)),
            full((1, 1)),
        ],
        out_specs=pl.BlockSpec((_GB, 1), lambda i: (i, 0)),
        out_shape=jax.ShapeDtypeStruct((_B, 1), jnp.float32),
    )(x, xpp, xr, xppr, am, ax, wl1, bl1, wg, bgs)


def kernel(x, edge_index, edge_attr, action, W1, b1, W2, b2, Wl1, bl1, Wg, bg):
    src = edge_index[0]
    dst = edge_index[1]

    gs, gd = _sc_gather(x, src, dst)
    w1ap = jnp.kron(jnp.eye(8, dtype=jnp.float32), W1[:_NODE])
    w1bp = jnp.kron(jnp.eye(8, dtype=jnp.float32), W1[_NODE:2 * _NODE])
    w1cp = jnp.kron(jnp.eye(32, dtype=jnp.float32), W1[2 * _NODE:])
    w2p = jnp.kron(jnp.eye(4, dtype=jnp.float32), W2)
    msg4 = _edge_mlp(gs.reshape(_E // 8, 128), gd.reshape(_E // 8, 128),
                     edge_attr.reshape(_E // 32, 128),
                     w1ap, w1bp, w1cp,
                     jnp.tile(b1, 4).reshape(1, 128), w2p,
                     jnp.tile(b2, 4).reshape(1, 128))
    xpp = _sc_scatter(src, msg4.reshape(_E, _HID))

    xr = jnp.roll(x.reshape(_B, _NNODES, _NODE), -1, axis=1).reshape(_N, _NODE)
    xppr = jnp.roll(xpp.reshape(_B, _NNODES, _HID), -1, axis=1).reshape(_N, _HID)
    am = action[:, :_NNODES].reshape(_N, 1)
    ax = jnp.concatenate(
        [jnp.zeros((_B, _NNODES - _NFACT), jnp.float32), action[:, _NNODES:]],
        axis=1).reshape(_N, 1)

    out = _head(x, xpp, xr, xppr, am, ax,
                Wl1, bl1.reshape(1, _HID), Wg, bg.reshape(1, 1))
    return out.reshape(_B)


# speedup vs baseline: 3.4690x; 1.0032x over previous
import functools

import jax
import jax.numpy as jnp
from jax import lax
from jax.experimental import pallas as pl
from jax.experimental.pallas import tpu as pltpu
from jax.experimental.pallas import tpu_sc as plsc

_NNODES = 50
_NODE = 16
_EATTR = 4
_HID = 32
_NFACT = 3
_B = 2000
_N = _B * _NNODES
_E = 1600000
_NGRP = _E // 128
_GPC = 20
_ECH = _GPC * 128
_NCH = _NGRP // _GPC
_NC = 2
_NS = 16
_HALF = _N // _NC
_ZR = 3128
_ACCR = _NS * _ZR
_DUMMY = _HALF + 8

_MESH = dict(core_axis_name="c", subcore_axis_name="s")


def _sc_gather(x, src, dst):
    def body(x_hbm, src_hbm, dst_hbm, gs_hbm, gd_hbm,
             idx_s, idx_d, buf_s, buf_d, sem_s, sem_d):
        c = lax.axis_index("c")
        s = lax.axis_index("s")
        w = c * _NS + s
        nq = jnp.where(w < _NCH % 32, _NCH // 32 + 1, _NCH // 32)

        def chunk(k, carry):
            q = w + k * 32
            e0 = q * _ECH
            pltpu.sync_copy(src_hbm.at[pl.ds(e0, _ECH)], idx_s)
            pltpu.sync_copy(dst_hbm.at[pl.ds(e0, _ECH)], idx_d)
            cps = [pltpu.async_copy(x_hbm.at[idx_s.at[pl.ds(j * 128, 128)]],
                                    buf_s.at[pl.ds(j * 128, 128)], sem_s)
                   for j in range(_GPC)]
            cpd = [pltpu.async_copy(x_hbm.at[idx_d.at[pl.ds(j * 128, 128)]],
                                    buf_d.at[pl.ds(j * 128, 128)], sem_d)
                   for j in range(_GPC)]
            for cp in cps:
                cp.wait()
            for cp in cpd:
                cp.wait()
            pltpu.sync_copy(buf_s, gs_hbm.at[pl.ds(e0, _ECH)])
            pltpu.sync_copy(buf_d, gd_hbm.at[pl.ds(e0, _ECH)])
            return carry

        lax.fori_loop(0, nq, chunk, 0)

    f = pl.kernel(
        body,
        mesh=plsc.VectorSubcoreMesh(**_MESH),
        compiler_params=pltpu.CompilerParams(use_tc_tiling_on_sc=False),
        out_type=[jax.ShapeDtypeStruct((_E, _NODE), jnp.float32),
                  jax.ShapeDtypeStruct((_E, _NODE), jnp.float32)],
        scratch_types=[
            pltpu.VMEM((_ECH,), jnp.int32),
            pltpu.VMEM((_ECH,), jnp.int32),
            pltpu.VMEM((_ECH, _NODE), jnp.float32),
            pltpu.VMEM((_ECH, _NODE), jnp.float32),
            pltpu.SemaphoreType.DMA,
            pltpu.SemaphoreType.DMA,
        ],
    )
    return f(x, src, dst)


_EB = 12800


def _edge_mlp_body(gs8, gd8, ea32, w1a, w1b, w1c, b1, w2, b2, out):
    r8 = _EB // 8
    r32 = _EB // 32
    r4 = _EB // 4
    t1 = jnp.dot(gs8[...], w1a[...], preferred_element_type=jnp.float32)
    t1 = t1.reshape(r8, 2, 128).reshape(r4, 128)
    t2 = jnp.dot(gd8[...], w1b[...], preferred_element_type=jnp.float32)
    t2 = t2.reshape(r8, 2, 128).reshape(r4, 128)
    t3 = jnp.dot(ea32[...], w1c[...], preferred_element_type=jnp.float32)
    t3 = t3.reshape(r32, 8, 128).reshape(r4, 128)
    h = jnp.maximum(t1 + t2 + t3 + b1[...], 0.0)
    out[...] = jnp.dot(h, w2[...], preferred_element_type=jnp.float32) + b2[...]


def _edge_mlp(gs8, gd8, ea32, w1a, w1b, w1c, b1, w2, b2):
    nsteps = _E // _EB
    full = lambda shape: pl.BlockSpec(shape, lambda i: (0, 0))
    return pl.pallas_call(
        _edge_mlp_body,
        grid=(nsteps,),
        in_specs=[
            pl.BlockSpec((_EB // 8, 128), lambda i: (i, 0)),
            pl.BlockSpec((_EB // 8, 128), lambda i: (i, 0)),
            pl.BlockSpec((_EB // 32, 128), lambda i: (i, 0)),
            full((128, 256)),
            full((128, 256)),
            full((128, 1024)),
            full((1, 128)),
            full((128, 128)),
            full((1, 128)),
        ],
        out_specs=pl.BlockSpec((_EB // 4, 128), lambda i: (i, 0)),
        out_shape=jax.ShapeDtypeStruct((_E // 4, 128), jnp.float32),
    )(gs8, gd8, ea32, w1a, w1b, w1c, b1, w2, b2)


_GPS = 2
_ECS = _GPS * 128
_NCS = _NGRP // _GPS
_NFULL = (_NCS // _NS) & ~1
_NTAIL = _NCS - _NFULL * _NS


def _sc_scatter(src, msg):
    def body(src_hbm, msg_hbm, xpp_hbm,
             src_v0, src_v1, idx_v0, idx_v1, msg_v0, msg_v1,
             acc, sem0, sem1):
        c = lax.axis_index("c")
        s = lax.axis_index("s")
        node_base = c * _HALF
        src_v = (src_v0, src_v1)
        idx_v = (idx_v0, idx_v1)
        msg_v = (msg_v0, msg_v1)
        sem = (sem0, sem1)

        def zrow(r, carry):
            z = jnp.zeros((16,), jnp.float32)
            msg_v0[r, pl.ds(0, 16)] = z
            msg_v0[r, pl.ds(16, 16)] = z
            return carry
        lax.fori_loop(0, _ECS, zrow, 0)
        a0 = s * _ZR
        nz = _ZR // _ECS
        for t in range(nz):
            pltpu.sync_copy(msg_v0, acc.at[pl.ds(a0 + t * _ECS, _ECS)])
        pltpu.sync_copy(msg_v0.at[pl.ds(0, _ZR - nz * _ECS)],
                        acc.at[pl.ds(a0 + nz * _ECS, _ZR - nz * _ECS)])
        plsc.subcore_barrier()

        def start_load(b, kk):
            q = s + jnp.minimum(kk, _NFULL - 1) * _NS
            e0 = q * _ECS
            pltpu.async_copy(src_hbm.at[pl.ds(e0, _ECS)], src_v[b], sem[b])
            pltpu.async_copy(msg_hbm.at[pl.ds(e0, _ECS)], msg_v[b], sem[b])

        def wait_load(b):
            pltpu.make_async_copy(src_hbm.at[pl.ds(0, _ECS)], src_v[b],
                                  sem[b]).wait()
            pltpu.make_async_copy(msg_hbm.at[pl.ds(0, _ECS)], msg_v[b],
                                  sem[b]).wait()

        def process(b):
            def cidx(i, carry2):
                j = i // 8
                o = (i % 8) * 16
                sv = src_v[b][pl.ds(i * 16, 16)]
                loc = sv - node_base
                ok = (loc >= 0) & (loc < _HALF)
                idx_v[b][j, pl.ds(o, 16)] = jnp.where(ok, loc, _DUMMY)
                return carry2
            lax.fori_loop(0, _GPS * 8, cidx, 0)
            for j in range(_GPS):
                pltpu.sync_copy(msg_v[b].at[pl.ds(j * 128, 128)],
                                acc.at[idx_v[b].at[j]], add=True)

        start_load(0, 0)
        start_load(1, 1)

        def pair(k, carry):
            for b in range(2):
                kk = 2 * k + b
                wait_load(b)
                process(b)
                start_load(b, kk + 2)
            return carry
        lax.fori_loop(0, _NFULL // 2, pair, 0)
        wait_load(0)
        wait_load(1)

        @pl.when(s < _NTAIL)
        def _():
            q = _NFULL * _NS + s
            e0 = q * _ECS
            pltpu.sync_copy(src_hbm.at[pl.ds(e0, _ECS)], src_v0)
            pltpu.sync_copy(msg_hbm.at[pl.ds(e0, _ECS)], msg_v0)
            process(0)

        plsc.subcore_barrier()

        last = _HALF - (_NS - 1) * _ZR

        @pl.when(s < _NS - 1)
        def _():
            pltpu.sync_copy(acc.at[pl.ds(a0, _ZR)],
                            xpp_hbm.at[pl.ds(node_base + a0, _ZR)])

        @pl.when(s == _NS - 1)
        def _():
            b = (_NS - 1) * _ZR
            pltpu.sync_copy(acc.at[pl.ds(b, last)],
                            xpp_hbm.at[pl.ds(node_base + b, last)])

    f = pl.kernel(
        body,
        mesh=plsc.VectorSubcoreMesh(**_MESH),
        compiler_params=pltpu.CompilerParams(use_tc_tiling_on_sc=False),
        out_type=jax.ShapeDtypeStruct((_N, _HID), jnp.float32),
        scratch_types=[
            pltpu.VMEM((_ECS,), jnp.int32),
            pltpu.VMEM((_ECS,), jnp.int32),
            pltpu.VMEM((_GPS, 128), jnp.int32),
            pltpu.VMEM((_GPS, 128), jnp.int32),
            pltpu.VMEM((_ECS, _HID), jnp.float32),
            pltpu.VMEM((_ECS, _HID), jnp.float32),
            pltpu.VMEM_SHARED((_ACCR, _HID), jnp.float32),
            pltpu.SemaphoreType.DMA,
            pltpu.SemaphoreType.DMA,
        ],
    )
    return f(src, msg)


_GB = 40
_RB = _GB * _NNODES


def _head_body(x, xpp, xr, xppr, am, ax, wl1, bl1, wg, bgs, out):
    wa1 = wl1[0:16, :]
    wa2 = wl1[16:48, :]
    wb1 = wl1[48:64, :]
    wb2 = wl1[64:96, :]
    wact = wl1[96:97, :]
    u = (jnp.dot(x[...], wa1, preferred_element_type=jnp.float32)
         + jnp.dot(xpp[...], wa2, preferred_element_type=jnp.float32))
    vr = (jnp.dot(xr[...], wb1, preferred_element_type=jnp.float32)
          + jnp.dot(xppr[...], wb2, preferred_element_type=jnp.float32))
    v = (jnp.dot(x[...], wb1, preferred_element_type=jnp.float32)
         + jnp.dot(xpp[...], wb2, preferred_element_type=jnp.float32))
    h1 = jnp.maximum(u + vr + am[...] * wact + bl1[...], 0.0)
    q = jnp.dot(h1, wg[...], preferred_element_type=jnp.float32)
    kmod = lax.broadcasted_iota(jnp.int32, (_RB, 1), 0) % _NNODES
    h2 = jnp.maximum(u + v + ax[...] * wact + bl1[...], 0.0)
    q2 = jnp.dot(h2, wg[...], preferred_element_type=jnp.float32)
    q = q + jnp.where(kmod >= _NNODES - _NFACT, q2, 0.0)
    gsel = (lax.broadcasted_iota(jnp.int32, (_GB, _RB), 0)
            == lax.broadcasted_iota(jnp.int32, (_GB, _RB), 1) // _NNODES)
    out[...] = (jnp.dot(gsel.astype(jnp.float32), q,
                        preferred_element_type=jnp.float32)
                + (_NNODES + _NFACT) * bgs[...])


def _head(x, xpp, xr, xppr, am, ax, wl1, bl1, wg, bgs):
    nsteps = _B // _GB
    full = lambda shape: pl.BlockSpec(shape, lambda i: (0, 0))
    return pl.pallas_call(
        _head_body,
        grid=(nsteps,),
        in_specs=[
            pl.BlockSpec((_RB, _NODE), lambda i: (i, 0)),
            pl.BlockSpec((_RB, _HID), lambda i: (i, 0)),
            pl.BlockSpec((_RB, _NODE), lambda i: (i, 0)),
            pl.BlockSpec((_RB, _HID), lambda i: (i, 0)),
            pl.BlockSpec((_RB, 1), lambda i: (i, 0)),
            pl.BlockSpec((_RB, 1), lambda i: (i, 0)),
            full((2 * (_NODE + _HID) + 1, _HID)),
            full((1, _HID)),
            full((_HID, 1)),
            full((1, 1)),
        ],
        out_specs=pl.BlockSpec((_GB, 1), lambda i: (i, 0)),
        out_shape=jax.ShapeDtypeStruct((_B, 1), jnp.float32),
    )(x, xpp, xr, xppr, am, ax, wl1, bl1, wg, bgs)


def kernel(x, edge_index, edge_attr, action, W1, b1, W2, b2, Wl1, bl1, Wg, bg):
    src = edge_index[0]
    dst = edge_index[1]

    gs, gd = _sc_gather(x, src, dst)
    w1ap = jnp.kron(jnp.eye(8, dtype=jnp.float32), W1[:_NODE])
    w1bp = jnp.kron(jnp.eye(8, dtype=jnp.float32), W1[_NODE:2 * _NODE])
    w1cp = jnp.kron(jnp.eye(32, dtype=jnp.float32), W1[2 * _NODE:])
    w2p = jnp.kron(jnp.eye(4, dtype=jnp.float32), W2)
    msg4 = _edge_mlp(gs.reshape(_E // 8, 128), gd.reshape(_E // 8, 128),
                     edge_attr.reshape(_E // 32, 128),
                     w1ap, w1bp, w1cp,
                     jnp.tile(b1, 4).reshape(1, 128), w2p,
                     jnp.tile(b2, 4).reshape(1, 128))
    xpp = _sc_scatter(src, msg4.reshape(_E, _HID))

    xr = jnp.roll(x.reshape(_B, _NNODES, _NODE), -1, axis=1).reshape(_N, _NODE)
    xppr = jnp.roll(xpp.reshape(_B, _NNODES, _HID), -1, axis=1).reshape(_N, _HID)
    am = action[:, :_NNODES].reshape(_N, 1)
    ax = jnp.concatenate(
        [jnp.zeros((_B, _NNODES - _NFACT), jnp.float32), action[:, _NNODES:]],
        axis=1).reshape(_N, 1)

    out = _head(x, xpp, xr, xppr, am, ax,
                Wl1, bl1.reshape(1, _HID), Wg, bg.reshape(1, 1))
    return out.reshape(_B)


# speedup vs baseline: 3.7897x; 1.0925x over previous
import functools

import jax
import jax.numpy as jnp
from jax import lax
from jax.experimental import pallas as pl
from jax.experimental.pallas import tpu as pltpu
from jax.experimental.pallas import tpu_sc as plsc

_NNODES = 50
_NODE = 16
_EATTR = 4
_HID = 32
_NFACT = 3
_B = 2000
_N = _B * _NNODES
_E = 1600000
_NGRP = _E // 128
_GPC = 20
_ECH = _GPC * 128
_NCH = _NGRP // _GPC
_NC = 2
_NS = 16
_HALF = _N // _NC
_ZR = 3128
_ACCR = _NS * _ZR
_DUMMY = _HALF + 8

_MESH = dict(core_axis_name="c", subcore_axis_name="s")


def _sc_gather(x, src, dst):
    def body(x_hbm, src_hbm, dst_hbm, gs_hbm, gd_hbm,
             idx_s, idx_d, buf_s, buf_d, sem_s, sem_d):
        c = lax.axis_index("c")
        s = lax.axis_index("s")
        w = c * _NS + s
        nq = jnp.where(w < _NCH % 32, _NCH // 32 + 1, _NCH // 32)

        def chunk(k, carry):
            q = w + k * 32
            e0 = q * _ECH
            pltpu.sync_copy(src_hbm.at[pl.ds(e0, _ECH)], idx_s)
            pltpu.sync_copy(dst_hbm.at[pl.ds(e0, _ECH)], idx_d)
            cps = [pltpu.async_copy(x_hbm.at[idx_s.at[pl.ds(j * 128, 128)]],
                                    buf_s.at[pl.ds(j * 128, 128)], sem_s)
                   for j in range(_GPC)]
            cpd = [pltpu.async_copy(x_hbm.at[idx_d.at[pl.ds(j * 128, 128)]],
                                    buf_d.at[pl.ds(j * 128, 128)], sem_d)
                   for j in range(_GPC)]
            for cp in cps:
                cp.wait()
            for cp in cpd:
                cp.wait()
            pltpu.sync_copy(buf_s, gs_hbm.at[pl.ds(e0, _ECH)])
            pltpu.sync_copy(buf_d, gd_hbm.at[pl.ds(e0, _ECH)])
            return carry

        lax.fori_loop(0, nq, chunk, 0)

    f = pl.kernel(
        body,
        mesh=plsc.VectorSubcoreMesh(**_MESH),
        compiler_params=pltpu.CompilerParams(use_tc_tiling_on_sc=False),
        out_type=[jax.ShapeDtypeStruct((_E, _NODE), jnp.float32),
                  jax.ShapeDtypeStruct((_E, _NODE), jnp.float32)],
        scratch_types=[
            pltpu.VMEM((_ECH,), jnp.int32),
            pltpu.VMEM((_ECH,), jnp.int32),
            pltpu.VMEM((_ECH, _NODE), jnp.float32),
            pltpu.VMEM((_ECH, _NODE), jnp.float32),
            pltpu.SemaphoreType.DMA,
            pltpu.SemaphoreType.DMA,
        ],
    )
    return f(x, src, dst)


_EB = 12800


def _edge_mlp_body(gs8, gd8, ea32, w1a, w1b, w1c, b1, w2, b2, out):
    r8 = _EB // 8
    r32 = _EB // 32
    r4 = _EB // 4
    t1 = jnp.dot(gs8[...], w1a[...], preferred_element_type=jnp.float32)
    t1 = t1.reshape(r8, 2, 128).reshape(r4, 128)
    t2 = jnp.dot(gd8[...], w1b[...], preferred_element_type=jnp.float32)
    t2 = t2.reshape(r8, 2, 128).reshape(r4, 128)
    t3 = jnp.dot(ea32[...], w1c[...], preferred_element_type=jnp.float32)
    t3 = t3.reshape(r32, 8, 128).reshape(r4, 128)
    h = jnp.maximum(t1 + t2 + t3 + b1[...], 0.0)
    msg = jnp.dot(h, w2[...], preferred_element_type=jnp.float32) + b2[...]
    out[...] = msg.reshape(r4 * 128)


def _edge_mlp(gs8, gd8, ea32, w1a, w1b, w1c, b1, w2, b2):
    nsteps = _E // _EB
    full = lambda shape: pl.BlockSpec(shape, lambda i: (0, 0))
    return pl.pallas_call(
        _edge_mlp_body,
        grid=(nsteps,),
        in_specs=[
            pl.BlockSpec((_EB // 8, 128), lambda i: (i, 0)),
            pl.BlockSpec((_EB // 8, 128), lambda i: (i, 0)),
            pl.BlockSpec((_EB // 32, 128), lambda i: (i, 0)),
            full((128, 256)),
            full((128, 256)),
            full((128, 1024)),
            full((1, 128)),
            full((128, 128)),
            full((1, 128)),
        ],
        out_specs=pl.BlockSpec((_EB * _HID,), lambda i: (i,)),
        out_shape=jax.ShapeDtypeStruct((_E * _HID,), jnp.float32),
    )(gs8, gd8, ea32, w1a, w1b, w1c, b1, w2, b2)


_GPS = 2
_ECS = _GPS * 128
_NCS = _NGRP // _GPS
_NFULL = (_NCS // _NS) & ~1
_NTAIL = _NCS - _NFULL * _NS


def _sc_scatter(src, msg):
    def body(src_hbm, msg_hbm, xpp_hbm,
             src_v0, src_v1, idx_v0, idx_v1, msg_v0, msg_v1,
             acc, sem0, sem1):
        c = lax.axis_index("c")
        s = lax.axis_index("s")
        node_base = c * _HALF
        src_v = (src_v0, src_v1)
        idx_v = (idx_v0, idx_v1)
        msg_v = (msg_v0, msg_v1)
        sem = (sem0, sem1)

        def zrow(r, carry):
            z = jnp.zeros((16,), jnp.float32)
            msg_v0[r, pl.ds(0, 16)] = z
            msg_v0[r, pl.ds(16, 16)] = z
            return carry
        lax.fori_loop(0, _ECS, zrow, 0)
        a0 = s * _ZR
        nz = _ZR // _ECS
        for t in range(nz):
            pltpu.sync_copy(msg_v0, acc.at[pl.ds(a0 + t * _ECS, _ECS)])
        pltpu.sync_copy(msg_v0.at[pl.ds(0, _ZR - nz * _ECS)],
                        acc.at[pl.ds(a0 + nz * _ECS, _ZR - nz * _ECS)])
        plsc.subcore_barrier()

        def start_load(b, kk):
            q = s + jnp.minimum(kk, _NFULL - 1) * _NS
            e0 = q * _ECS
            pltpu.async_copy(src_hbm.at[pl.ds(e0, _ECS)], src_v[b], sem[b])
            pltpu.async_copy(msg_hbm.at[pl.ds(e0, _ECS)], msg_v[b], sem[b])

        def wait_load(b):
            pltpu.make_async_copy(src_hbm.at[pl.ds(0, _ECS)], src_v[b],
                                  sem[b]).wait()
            pltpu.make_async_copy(msg_hbm.at[pl.ds(0, _ECS)], msg_v[b],
                                  sem[b]).wait()

        def process(b):
            def cidx(i, carry2):
                j = i // 8
                o = (i % 8) * 16
                sv = src_v[b][pl.ds(i * 16, 16)]
                loc = sv - node_base
                ok = (loc >= 0) & (loc < _HALF)
                idx_v[b][j, pl.ds(o, 16)] = jnp.where(ok, loc, _DUMMY)
                return carry2
            lax.fori_loop(0, _GPS * 8, cidx, 0)
            for j in range(_GPS):
                pltpu.sync_copy(msg_v[b].at[pl.ds(j * 128, 128)],
                                acc.at[idx_v[b].at[j]], add=True)

        start_load(0, 0)
        start_load(1, 1)

        def pair(k, carry):
            for b in range(2):
                kk = 2 * k + b
                wait_load(b)
                process(b)
                start_load(b, kk + 2)
            return carry
        lax.fori_loop(0, _NFULL // 2, pair, 0)
        wait_load(0)
        wait_load(1)

        @pl.when(s < _NTAIL)
        def _():
            q = _NFULL * _NS + s
            e0 = q * _ECS
            pltpu.sync_copy(src_hbm.at[pl.ds(e0, _ECS)], src_v0)
            pltpu.sync_copy(msg_hbm.at[pl.ds(e0, _ECS)], msg_v0)
            process(0)

        plsc.subcore_barrier()

        last = _HALF - (_NS - 1) * _ZR

        @pl.when(s < _NS - 1)
        def _():
            pltpu.sync_copy(acc.at[pl.ds(a0, _ZR)],
                            xpp_hbm.at[pl.ds(node_base + a0, _ZR)])

        @pl.when(s == _NS - 1)
        def _():
            b = (_NS - 1) * _ZR
            pltpu.sync_copy(acc.at[pl.ds(b, last)],
                            xpp_hbm.at[pl.ds(node_base + b, last)])

    f = pl.kernel(
        body,
        mesh=plsc.VectorSubcoreMesh(**_MESH),
        compiler_params=pltpu.CompilerParams(use_tc_tiling_on_sc=False),
        out_type=jax.ShapeDtypeStruct((_N, _HID), jnp.float32),
        scratch_types=[
            pltpu.VMEM((_ECS,), jnp.int32),
            pltpu.VMEM((_ECS,), jnp.int32),
            pltpu.VMEM((_GPS, 128), jnp.int32),
            pltpu.VMEM((_GPS, 128), jnp.int32),
            pltpu.VMEM((_ECS, _HID), jnp.float32),
            pltpu.VMEM((_ECS, _HID), jnp.float32),
            pltpu.VMEM_SHARED((_ACCR, _HID), jnp.float32),
            pltpu.SemaphoreType.DMA,
            pltpu.SemaphoreType.DMA,
        ],
    )
    return f(src, msg)


_GB = 40
_RB = _GB * _NNODES


def _head_body(x, xpp, am, ax, wl1, bl1, wg, bgs, out):
    wa1 = wl1[0:16, :]
    wa2 = wl1[16:48, :]
    wb1 = wl1[48:64, :]
    wb2 = wl1[64:96, :]
    wact = wl1[96:97, :]
    u = (jnp.dot(x[...], wa1, preferred_element_type=jnp.float32)
         + jnp.dot(xpp[...], wa2, preferred_element_type=jnp.float32))
    v = (jnp.dot(x[...], wb1, preferred_element_type=jnp.float32)
         + jnp.dot(xpp[...], wb2, preferred_element_type=jnp.float32))
    kroll = lax.broadcasted_iota(jnp.int32, (_RB, 1), 0) % _NNODES
    vr = jnp.where(kroll == _NNODES - 1,
                   jnp.roll(v, _NNODES - 1, axis=0),
                   jnp.roll(v, -1, axis=0))
    h1 = jnp.maximum(u + vr + am[...] * wact + bl1[...], 0.0)
    q = jnp.dot(h1, wg[...], preferred_element_type=jnp.float32)
    kmod = lax.broadcasted_iota(jnp.int32, (_RB, 1), 0) % _NNODES
    h2 = jnp.maximum(u + v + ax[...] * wact + bl1[...], 0.0)
    q2 = jnp.dot(h2, wg[...], preferred_element_type=jnp.float32)
    q = q + jnp.where(kmod >= _NNODES - _NFACT, q2, 0.0)
    gsel = (lax.broadcasted_iota(jnp.int32, (_GB, _RB), 0)
            == lax.broadcasted_iota(jnp.int32, (_GB, _RB), 1) // _NNODES)
    out[...] = (jnp.dot(gsel.astype(jnp.float32), q,
                        preferred_element_type=jnp.float32)
                + (_NNODES + _NFACT) * bgs[...])


def _head(x, xpp, am, ax, wl1, bl1, wg, bgs):
    nsteps = _B // _GB
    full = lambda shape: pl.BlockSpec(shape, lambda i: (0, 0))
    return pl.pallas_call(
        _head_body,
        grid=(nsteps,),
        in_specs=[
            pl.BlockSpec((_RB, _NODE), lambda i: (i, 0)),
            pl.BlockSpec((_RB, _HID), lambda i: (i, 0)),
            pl.BlockSpec((_RB, 1), lambda i: (i, 0)),
            pl.BlockSpec((_RB, 1), lambda i: (i, 0)),
            full((2 * (_NODE + _HID) + 1, _HID)),
            full((1, _HID)),
            full((_HID, 1)),
            full((1, 1)),
        ],
        out_specs=pl.BlockSpec((_GB, 1), lambda i: (i, 0)),
        out_shape=jax.ShapeDtypeStruct((_B, 1), jnp.float32),
    )(x, xpp, am, ax, wl1, bl1, wg, bgs)


def kernel(x, edge_index, edge_attr, action, W1, b1, W2, b2, Wl1, bl1, Wg, bg):
    src = edge_index[0]
    dst = edge_index[1]

    gs, gd = _sc_gather(x, src, dst)
    w1ap = jnp.kron(jnp.eye(8, dtype=jnp.float32), W1[:_NODE])
    w1bp = jnp.kron(jnp.eye(8, dtype=jnp.float32), W1[_NODE:2 * _NODE])
    w1cp = jnp.kron(jnp.eye(32, dtype=jnp.float32), W1[2 * _NODE:])
    w2p = jnp.kron(jnp.eye(4, dtype=jnp.float32), W2)
    msgf = _edge_mlp(gs.reshape(_E // 8, 128), gd.reshape(_E // 8, 128),
                     edge_attr.reshape(_E // 32, 128),
                     w1ap, w1bp, w1cp,
                     jnp.tile(b1, 4).reshape(1, 128), w2p,
                     jnp.tile(b2, 4).reshape(1, 128))
    xpp = _sc_scatter(src, msgf.reshape(_E, _HID))

    am = action[:, :_NNODES].reshape(_N, 1)
    ax = jnp.concatenate(
        [jnp.zeros((_B, _NNODES - _NFACT), jnp.float32), action[:, _NNODES:]],
        axis=1).reshape(_N, 1)

    out = _head(x, xpp, am, ax,
                Wl1, bl1.reshape(1, _HID), Wg, bg.reshape(1, 1))
    return out.reshape(_B)


# speedup vs baseline: 6.0549x; 1.5977x over previous
import functools

import jax
import jax.numpy as jnp
from jax import lax
from jax.experimental import pallas as pl
from jax.experimental.pallas import tpu as pltpu
from jax.experimental.pallas import tpu_sc as plsc

_NNODES = 50
_NODE = 16
_EATTR = 4
_HID = 32
_NFACT = 3
_B = 2000
_N = _B * _NNODES
_E = 1600000
_NGRP = _E // 128
_GPC = 20
_ECH = _GPC * 128
_NCH = _NGRP // _GPC
_NC = 2
_NS = 16
_HALF = _N // _NC
_ZR = 3128
_ACCR = _NS * _ZR
_DUMMY = _HALF + 8

_MESH = dict(core_axis_name="c", subcore_axis_name="s")


def _sc_gather(x, src, dst):
    def body(x_hbm, src_hbm, dst_hbm, gs_hbm, gd_hbm,
             idx_s, idx_d, buf_s, buf_d, sem_s, sem_d):
        c = lax.axis_index("c")
        s = lax.axis_index("s")
        w = c * _NS + s
        nq = jnp.where(w < _NCH % 32, _NCH // 32 + 1, _NCH // 32)

        def chunk(k, carry):
            q = w + k * 32
            e0 = q * _ECH
            pltpu.sync_copy(src_hbm.at[pl.ds(e0, _ECH)], idx_s)
            pltpu.sync_copy(dst_hbm.at[pl.ds(e0, _ECH)], idx_d)
            cps = [pltpu.async_copy(x_hbm.at[idx_s.at[pl.ds(j * 128, 128)]],
                                    buf_s.at[pl.ds(j * 128, 128)], sem_s)
                   for j in range(_GPC)]
            cpd = [pltpu.async_copy(x_hbm.at[idx_d.at[pl.ds(j * 128, 128)]],
                                    buf_d.at[pl.ds(j * 128, 128)], sem_d)
                   for j in range(_GPC)]
            for cp in cps:
                cp.wait()
            for cp in cpd:
                cp.wait()
            pltpu.sync_copy(buf_s, gs_hbm.at[pl.ds(e0, _ECH)])
            pltpu.sync_copy(buf_d, gd_hbm.at[pl.ds(e0, _ECH)])
            return carry

        lax.fori_loop(0, nq, chunk, 0)

    f = pl.kernel(
        body,
        mesh=plsc.VectorSubcoreMesh(**_MESH),
        compiler_params=pltpu.CompilerParams(use_tc_tiling_on_sc=False),
        out_type=[jax.ShapeDtypeStruct((_E, _NODE), jnp.float32),
                  jax.ShapeDtypeStruct((_E, _NODE), jnp.float32)],
        scratch_types=[
            pltpu.VMEM((_ECH,), jnp.int32),
            pltpu.VMEM((_ECH,), jnp.int32),
            pltpu.VMEM((_ECH, _NODE), jnp.float32),
            pltpu.VMEM((_ECH, _NODE), jnp.float32),
            pltpu.SemaphoreType.DMA,
            pltpu.SemaphoreType.DMA,
        ],
    )
    return f(x, src, dst)


_EB = 12800


def _edge_mlp_body(gs8, gd8, ea32, w1a, w1b, w1c, b1, w2, b2, out):
    r8 = _EB // 8
    r32 = _EB // 32
    r4 = _EB // 4
    t1 = jnp.dot(gs8[...], w1a[...], preferred_element_type=jnp.float32)
    t1 = t1.reshape(r8, 2, 128).reshape(r4, 128)
    t2 = jnp.dot(gd8[...], w1b[...], preferred_element_type=jnp.float32)
    t2 = t2.reshape(r8, 2, 128).reshape(r4, 128)
    t3 = jnp.dot(ea32[...], w1c[...], preferred_element_type=jnp.float32)
    t3 = t3.reshape(r32, 8, 128).reshape(r4, 128)
    h = jnp.maximum(t1 + t2 + t3 + b1[...], 0.0)
    msg = jnp.dot(h, w2[...], preferred_element_type=jnp.float32) + b2[...]
    out[...] = msg.reshape(r4 * 128)


def _edge_mlp(gs8, gd8, ea32, w1a, w1b, w1c, b1, w2, b2):
    nsteps = _E // _EB
    full = lambda shape: pl.BlockSpec(shape, lambda i: (0, 0))
    return pl.pallas_call(
        _edge_mlp_body,
        grid=(nsteps,),
        in_specs=[
            pl.BlockSpec((_EB // 8, 128), lambda i: (i, 0)),
            pl.BlockSpec((_EB // 8, 128), lambda i: (i, 0)),
            pl.BlockSpec((_EB // 32, 128), lambda i: (i, 0)),
            full((128, 256)),
            full((128, 256)),
            full((128, 1024)),
            full((1, 128)),
            full((128, 128)),
            full((1, 128)),
        ],
        out_specs=pl.BlockSpec((_EB * _HID,), lambda i: (i,)),
        out_shape=jax.ShapeDtypeStruct((_E * _HID,), jnp.float32),
    )(gs8, gd8, ea32, w1a, w1b, w1c, b1, w2, b2)


_GPS = 2
_ECS = _GPS * 128
_NCS = _NGRP // _GPS
_NFULL = (_NCS // _NS) & ~1
_NTAIL = _NCS - _NFULL * _NS


def _sc_scatter(src, msg):
    def body(src_hbm, msg_hbm, xpp_hbm,
             src_v0, src_v1, idx_v0, idx_v1, msg_v0, msg_v1,
             acc, sem0, sem1):
        c = lax.axis_index("c")
        s = lax.axis_index("s")
        node_base = c * _HALF
        src_v = (src_v0, src_v1)
        idx_v = (idx_v0, idx_v1)
        msg_v = (msg_v0, msg_v1)
        sem = (sem0, sem1)

        def zrow(r, carry):
            z = jnp.zeros((16,), jnp.float32)
            msg_v0[r, pl.ds(0, 16)] = z
            msg_v0[r, pl.ds(16, 16)] = z
            return carry
        lax.fori_loop(0, _ECS, zrow, 0)
        a0 = s * _ZR
        nz = _ZR // _ECS
        for t in range(nz):
            pltpu.sync_copy(msg_v0, acc.at[pl.ds(a0 + t * _ECS, _ECS)])
        pltpu.sync_copy(msg_v0.at[pl.ds(0, _ZR - nz * _ECS)],
                        acc.at[pl.ds(a0 + nz * _ECS, _ZR - nz * _ECS)])
        plsc.subcore_barrier()

        def start_load(b, kk):
            q = s + jnp.minimum(kk, _NFULL - 1) * _NS
            e0 = q * _ECS
            pltpu.async_copy(src_hbm.at[pl.ds(e0, _ECS)], src_v[b], sem[b])
            pltpu.async_copy(msg_hbm.at[pl.ds(e0, _ECS)], msg_v[b], sem[b])

        def wait_load(b):
            pltpu.make_async_copy(src_hbm.at[pl.ds(0, _ECS)], src_v[b],
                                  sem[b]).wait()
            pltpu.make_async_copy(msg_hbm.at[pl.ds(0, _ECS)], msg_v[b],
                                  sem[b]).wait()

        def process(b):
            def cidx(i, carry2):
                j = i // 8
                o = (i % 8) * 16
                sv = src_v[b][pl.ds(i * 16, 16)]
                loc = sv - node_base
                ok = (loc >= 0) & (loc < _HALF)
                idx_v[b][j, pl.ds(o, 16)] = jnp.where(ok, loc, _DUMMY)
                return carry2
            lax.fori_loop(0, _GPS * 8, cidx, 0)
            for j in range(_GPS):
                pltpu.sync_copy(msg_v[b].at[pl.ds(j * 128, 128)],
                                acc.at[idx_v[b].at[j]], add=True)

        start_load(0, 0)
        start_load(1, 1)

        def pair(k, carry):
            for b in range(2):
                kk = 2 * k + b
                wait_load(b)
                process(b)
                start_load(b, kk + 2)
            return carry
        lax.fori_loop(0, _NFULL // 2, pair, 0)
        wait_load(0)
        wait_load(1)

        @pl.when(s < _NTAIL)
        def _():
            q = _NFULL * _NS + s
            e0 = q * _ECS
            pltpu.sync_copy(src_hbm.at[pl.ds(e0, _ECS)], src_v0)
            pltpu.sync_copy(msg_hbm.at[pl.ds(e0, _ECS)], msg_v0)
            process(0)

        plsc.subcore_barrier()

        last = _HALF - (_NS - 1) * _ZR

        @pl.when(s < _NS - 1)
        def _():
            pltpu.sync_copy(acc.at[pl.ds(a0, _ZR)],
                            xpp_hbm.at[pl.ds(node_base + a0, _ZR)])

        @pl.when(s == _NS - 1)
        def _():
            b = (_NS - 1) * _ZR
            pltpu.sync_copy(acc.at[pl.ds(b, last)],
                            xpp_hbm.at[pl.ds(node_base + b, last)])

    f = pl.kernel(
        body,
        mesh=plsc.VectorSubcoreMesh(**_MESH),
        compiler_params=pltpu.CompilerParams(use_tc_tiling_on_sc=False),
        out_type=jax.ShapeDtypeStruct((_N, _HID), jnp.float32),
        scratch_types=[
            pltpu.VMEM((_ECS,), jnp.int32),
            pltpu.VMEM((_ECS,), jnp.int32),
            pltpu.VMEM((_GPS, 128), jnp.int32),
            pltpu.VMEM((_GPS, 128), jnp.int32),
            pltpu.VMEM((_ECS, _HID), jnp.float32),
            pltpu.VMEM((_ECS, _HID), jnp.float32),
            pltpu.VMEM_SHARED((_ACCR, _HID), jnp.float32),
            pltpu.SemaphoreType.DMA,
            pltpu.SemaphoreType.DMA,
        ],
    )
    return f(src, msg)


_GB = 40
_RB = _GB * _NNODES


def _head_body(x, xpp, am, ax, wl1, bl1, wg, bgs, out):
    wa1 = wl1[0:16, :]
    wa2 = wl1[16:48, :]
    wb1 = wl1[48:64, :]
    wb2 = wl1[64:96, :]
    wact = wl1[96:97, :]
    u = (jnp.dot(x[...], wa1, preferred_element_type=jnp.float32)
         + jnp.dot(xpp[...], wa2, preferred_element_type=jnp.float32))
    v = (jnp.dot(x[...], wb1, preferred_element_type=jnp.float32)
         + jnp.dot(xpp[...], wb2, preferred_element_type=jnp.float32))
    kroll = lax.broadcasted_iota(jnp.int32, (_RB, 1), 0) % _NNODES
    vr = jnp.where(kroll == _NNODES - 1,
                   jnp.roll(v, _NNODES - 1, axis=0),
                   jnp.roll(v, -1, axis=0))
    h1 = jnp.maximum(u + vr + am[...] * wact + bl1[...], 0.0)
    q = jnp.dot(h1, wg[...], preferred_element_type=jnp.float32)
    kmod = lax.broadcasted_iota(jnp.int32, (_RB, 1), 0) % _NNODES
    h2 = jnp.maximum(u + v + ax[...] * wact + bl1[...], 0.0)
    q2 = jnp.dot(h2, wg[...], preferred_element_type=jnp.float32)
    q = q + jnp.where(kmod >= _NNODES - _NFACT, q2, 0.0)
    gsel = (lax.broadcasted_iota(jnp.int32, (_GB, _RB), 0)
            == lax.broadcasted_iota(jnp.int32, (_GB, _RB), 1) // _NNODES)
    out[...] = (jnp.dot(gsel.astype(jnp.float32), q,
                        preferred_element_type=jnp.float32)
                + (_NNODES + _NFACT) * bgs[...])


def _head(x, xpp, am, ax, wl1, bl1, wg, bgs):
    nsteps = _B // _GB
    full = lambda shape: pl.BlockSpec(shape, lambda i: (0, 0))
    return pl.pallas_call(
        _head_body,
        grid=(nsteps,),
        in_specs=[
            pl.BlockSpec((_RB, _NODE), lambda i: (i, 0)),
            pl.BlockSpec((_RB, _HID), lambda i: (i, 0)),
            pl.BlockSpec((_RB, 1), lambda i: (i, 0)),
            pl.BlockSpec((_RB, 1), lambda i: (i, 0)),
            full((2 * (_NODE + _HID) + 1, _HID)),
            full((1, _HID)),
            full((_HID, 1)),
            full((1, 1)),
        ],
        out_specs=pl.BlockSpec((_GB, 1), lambda i: (i, 0)),
        out_shape=jax.ShapeDtypeStruct((_B, 1), jnp.float32),
    )(x, xpp, am, ax, wl1, bl1, wg, bgs)


def kernel(x, edge_index, edge_attr, action, W1, b1, W2, b2, Wl1, bl1, Wg, bg):
    src = edge_index[0]
    dst = edge_index[1]

    gs, gd = _sc_gather(x, src, dst)
    w1ap = jnp.kron(jnp.eye(8, dtype=jnp.float32), W1[:_NODE])
    w1bp = jnp.kron(jnp.eye(8, dtype=jnp.float32), W1[_NODE:2 * _NODE])
    w1cp = jnp.kron(jnp.eye(32, dtype=jnp.float32), W1[2 * _NODE:])
    w2p = jnp.kron(jnp.eye(4, dtype=jnp.float32), W2)
    ea32 = (edge_attr.T.reshape(_EATTR, _E // 32, 32)
            .transpose(1, 2, 0).reshape(_E // 32, 128))
    msgf = _edge_mlp(gs.reshape(_E // 8, 128), gd.reshape(_E // 8, 128),
                     ea32,
                     w1ap, w1bp, w1cp,
                     jnp.tile(b1, 4).reshape(1, 128), w2p,
                     jnp.tile(b2, 4).reshape(1, 128))
    xpp = _sc_scatter(src, msgf.reshape(_E, _HID))

    am = action[:, :_NNODES].reshape(_N, 1)
    ax = jnp.concatenate(
        [jnp.zeros((_B, _NNODES - _NFACT), jnp.float32), action[:, _NNODES:]],
        axis=1).reshape(_N, 1)

    out = _head(x, xpp, am, ax,
                Wl1, bl1.reshape(1, _HID), Wg, bg.reshape(1, 1))
    return out.reshape(_B)


# speedup vs baseline: 8.4153x; 1.3898x over previous
import functools

import jax
import jax.numpy as jnp
from jax import lax
from jax.experimental import pallas as pl
from jax.experimental.pallas import tpu as pltpu
from jax.experimental.pallas import tpu_sc as plsc

_NNODES = 50
_NODE = 16
_EATTR = 4
_HID = 32
_NFACT = 3
_B = 2000
_N = _B * _NNODES
_E = 1600000
_NGRP = _E // 128
_GPC = 20
_ECH = _GPC * 128
_NCH = _NGRP // _GPC
_NC = 2
_NS = 16
_HALF = _N // _NC
_ZR = 3128
_ACCR = _NS * _ZR
_DUMMY = _HALF + 8

_MESH = dict(core_axis_name="c", subcore_axis_name="s")


def _sc_gather(x, src, dst):
    def body(x_hbm, src_hbm, dst_hbm, gs_hbm, gd_hbm,
             idx_s, idx_d, buf_s, buf_d, sem_s, sem_d):
        c = lax.axis_index("c")
        s = lax.axis_index("s")
        w = c * _NS + s
        nq = jnp.where(w < _NCH % 32, _NCH // 32 + 1, _NCH // 32)

        def chunk(k, carry):
            q = w + k * 32
            e0 = q * _ECH
            pltpu.sync_copy(src_hbm.at[pl.ds(e0, _ECH)], idx_s)
            pltpu.sync_copy(dst_hbm.at[pl.ds(e0, _ECH)], idx_d)
            cps = [pltpu.async_copy(x_hbm.at[idx_s.at[pl.ds(j * 128, 128)]],
                                    buf_s.at[pl.ds(j * 128, 128)], sem_s)
                   for j in range(_GPC)]
            cpd = [pltpu.async_copy(x_hbm.at[idx_d.at[pl.ds(j * 128, 128)]],
                                    buf_d.at[pl.ds(j * 128, 128)], sem_d)
                   for j in range(_GPC)]
            for cp in cps:
                cp.wait()
            for cp in cpd:
                cp.wait()
            pltpu.sync_copy(buf_s, gs_hbm.at[pl.ds(e0, _ECH)])
            pltpu.sync_copy(buf_d, gd_hbm.at[pl.ds(e0, _ECH)])
            return carry

        lax.fori_loop(0, nq, chunk, 0)

    f = pl.kernel(
        body,
        mesh=plsc.VectorSubcoreMesh(**_MESH),
        compiler_params=pltpu.CompilerParams(use_tc_tiling_on_sc=False),
        out_type=[jax.ShapeDtypeStruct((_E, _NODE), jnp.float32),
                  jax.ShapeDtypeStruct((_E, _NODE), jnp.float32)],
        scratch_types=[
            pltpu.VMEM((_ECH,), jnp.int32),
            pltpu.VMEM((_ECH,), jnp.int32),
            pltpu.VMEM((_ECH, _NODE), jnp.float32),
            pltpu.VMEM((_ECH, _NODE), jnp.float32),
            pltpu.SemaphoreType.DMA,
            pltpu.SemaphoreType.DMA,
        ],
    )
    return f(x, src, dst)


_EB = 12800


def _edge_mlp_body(gs8, gd8, ea32, w1a, w1b, w1c, b1, w2, b2, out):
    r8 = _EB // 8
    r32 = _EB // 32
    r4 = _EB // 4
    t1 = jnp.dot(gs8[...], w1a[...], preferred_element_type=jnp.float32)
    t1 = t1.reshape(r8, 2, 128).reshape(r4, 128)
    t2 = jnp.dot(gd8[...], w1b[...], preferred_element_type=jnp.float32)
    t2 = t2.reshape(r8, 2, 128).reshape(r4, 128)
    t3 = jnp.dot(ea32[...], w1c[...], preferred_element_type=jnp.float32)
    t3 = t3.reshape(r32, 8, 128).reshape(r4, 128)
    h = jnp.maximum(t1 + t2 + t3 + b1[...], 0.0)
    msg = jnp.dot(h, w2[...], preferred_element_type=jnp.float32) + b2[...]
    out[...] = msg.reshape(r4 * 128)


def _edge_mlp(gs8, gd8, ea32, w1a, w1b, w1c, b1, w2, b2):
    nsteps = _E // _EB
    full = lambda shape: pl.BlockSpec(shape, lambda i: (0, 0))
    return pl.pallas_call(
        _edge_mlp_body,
        grid=(nsteps,),
        in_specs=[
            pl.BlockSpec((_EB // 8, 128), lambda i: (i, 0)),
            pl.BlockSpec((_EB // 8, 128), lambda i: (i, 0)),
            pl.BlockSpec((_EB // 32, 128), lambda i: (i, 0)),
            full((128, 256)),
            full((128, 256)),
            full((128, 1024)),
            full((1, 128)),
            full((128, 128)),
            full((1, 128)),
        ],
        out_specs=pl.BlockSpec((_EB * _HID,), lambda i: (i,)),
        out_shape=jax.ShapeDtypeStruct((_E * _HID,), jnp.float32),
    )(gs8, gd8, ea32, w1a, w1b, w1c, b1, w2, b2)


_GPS = 2
_ECS = _GPS * 128
_NCS = _NGRP // _GPS
_NFULL = (_NCS // _NS) & ~1
_NTAIL = _NCS - _NFULL * _NS


def _sc_scatter(src, msg):
    def body(src_hbm, msg_hbm, xpp_hbm,
             src_v0, src_v1, idx_v0, idx_v1, msg_v0, msg_v1,
             acc, sem0, sem1):
        c = lax.axis_index("c")
        s = lax.axis_index("s")
        node_base = c * _HALF
        src_v = (src_v0, src_v1)
        idx_v = (idx_v0, idx_v1)
        msg_v = (msg_v0, msg_v1)
        sem = (sem0, sem1)

        def zrow(r, carry):
            z = jnp.zeros((16,), jnp.float32)
            msg_v0[r, pl.ds(0, 16)] = z
            msg_v0[r, pl.ds(16, 16)] = z
            return carry
        lax.fori_loop(0, _ECS, zrow, 0)
        a0 = s * _ZR
        nz = _ZR // _ECS
        for t in range(nz):
            pltpu.sync_copy(msg_v0, acc.at[pl.ds(a0 + t * _ECS, _ECS)])
        pltpu.sync_copy(msg_v0.at[pl.ds(0, _ZR - nz * _ECS)],
                        acc.at[pl.ds(a0 + nz * _ECS, _ZR - nz * _ECS)])
        plsc.subcore_barrier()

        def start_load(b, kk):
            q = s + jnp.minimum(kk, _NFULL - 1) * _NS
            e0 = q * _ECS
            pltpu.async_copy(src_hbm.at[pl.ds(e0, _ECS)], src_v[b], sem[b])
            pltpu.async_copy(msg_hbm.at[pl.ds(e0, _ECS)], msg_v[b], sem[b])

        def wait_load(b):
            pltpu.make_async_copy(src_hbm.at[pl.ds(0, _ECS)], src_v[b],
                                  sem[b]).wait()
            pltpu.make_async_copy(msg_hbm.at[pl.ds(0, _ECS)], msg_v[b],
                                  sem[b]).wait()

        dummy_rows = _DUMMY + lax.iota(jnp.int32, 16)

        def process(b):
            def cidx(i, carry2):
                j = i // 8
                o = (i % 8) * 16
                sv = src_v[b][pl.ds(i * 16, 16)]
                loc = sv - node_base
                ok = (loc >= 0) & (loc < _HALF)
                idx_v[b][j, pl.ds(o, 16)] = jnp.where(ok, loc, dummy_rows)
                return carry2
            lax.fori_loop(0, _GPS * 8, cidx, 0)
            for j in range(_GPS):
                pltpu.sync_copy(msg_v[b].at[pl.ds(j * 128, 128)],
                                acc.at[idx_v[b].at[j]], add=True)

        start_load(0, 0)
        start_load(1, 1)

        def pair(k, carry):
            for b in range(2):
                kk = 2 * k + b
                wait_load(b)
                process(b)
                start_load(b, kk + 2)
            return carry
        lax.fori_loop(0, _NFULL // 2, pair, 0)
        wait_load(0)
        wait_load(1)

        @pl.when(s < _NTAIL)
        def _():
            q = _NFULL * _NS + s
            e0 = q * _ECS
            pltpu.sync_copy(src_hbm.at[pl.ds(e0, _ECS)], src_v0)
            pltpu.sync_copy(msg_hbm.at[pl.ds(e0, _ECS)], msg_v0)
            process(0)

        plsc.subcore_barrier()

        last = _HALF - (_NS - 1) * _ZR

        @pl.when(s < _NS - 1)
        def _():
            pltpu.sync_copy(acc.at[pl.ds(a0, _ZR)],
                            xpp_hbm.at[pl.ds(node_base + a0, _ZR)])

        @pl.when(s == _NS - 1)
        def _():
            b = (_NS - 1) * _ZR
            pltpu.sync_copy(acc.at[pl.ds(b, last)],
                            xpp_hbm.at[pl.ds(node_base + b, last)])

    f = pl.kernel(
        body,
        mesh=plsc.VectorSubcoreMesh(**_MESH),
        compiler_params=pltpu.CompilerParams(use_tc_tiling_on_sc=False),
        out_type=jax.ShapeDtypeStruct((_N, _HID), jnp.float32),
        scratch_types=[
            pltpu.VMEM((_ECS,), jnp.int32),
            pltpu.VMEM((_ECS,), jnp.int32),
            pltpu.VMEM((_GPS, 128), jnp.int32),
            pltpu.VMEM((_GPS, 128), jnp.int32),
            pltpu.VMEM((_ECS, _HID), jnp.float32),
            pltpu.VMEM((_ECS, _HID), jnp.float32),
            pltpu.VMEM_SHARED((_ACCR, _HID), jnp.float32),
            pltpu.SemaphoreType.DMA,
            pltpu.SemaphoreType.DMA,
        ],
    )
    return f(src, msg)


_GB = 40
_RB = _GB * _NNODES


def _head_body(x, xpp, am, ax, wl1, bl1, wg, bgs, out):
    wa1 = wl1[0:16, :]
    wa2 = wl1[16:48, :]
    wb1 = wl1[48:64, :]
    wb2 = wl1[64:96, :]
    wact = wl1[96:97, :]
    u = (jnp.dot(x[...], wa1, preferred_element_type=jnp.float32)
         + jnp.dot(xpp[...], wa2, preferred_element_type=jnp.float32))
    v = (jnp.dot(x[...], wb1, preferred_element_type=jnp.float32)
         + jnp.dot(xpp[...], wb2, preferred_element_type=jnp.float32))
    kroll = lax.broadcasted_iota(jnp.int32, (_RB, 1), 0) % _NNODES
    vr = jnp.where(kroll == _NNODES - 1,
                   jnp.roll(v, _NNODES - 1, axis=0),
                   jnp.roll(v, -1, axis=0))
    h1 = jnp.maximum(u + vr + am[...] * wact + bl1[...], 0.0)
    q = jnp.dot(h1, wg[...], preferred_element_type=jnp.float32)
    kmod = lax.broadcasted_iota(jnp.int32, (_RB, 1), 0) % _NNODES
    h2 = jnp.maximum(u + v + ax[...] * wact + bl1[...], 0.0)
    q2 = jnp.dot(h2, wg[...], preferred_element_type=jnp.float32)
    q = q + jnp.where(kmod >= _NNODES - _NFACT, q2, 0.0)
    gsel = (lax.broadcasted_iota(jnp.int32, (_GB, _RB), 0)
            == lax.broadcasted_iota(jnp.int32, (_GB, _RB), 1) // _NNODES)
    out[...] = (jnp.dot(gsel.astype(jnp.float32), q,
                        preferred_element_type=jnp.float32)
                + (_NNODES + _NFACT) * bgs[...])


def _head(x, xpp, am, ax, wl1, bl1, wg, bgs):
    nsteps = _B // _GB
    full = lambda shape: pl.BlockSpec(shape, lambda i: (0, 0))
    return pl.pallas_call(
        _head_body,
        grid=(nsteps,),
        in_specs=[
            pl.BlockSpec((_RB, _NODE), lambda i: (i, 0)),
            pl.BlockSpec((_RB, _HID), lambda i: (i, 0)),
            pl.BlockSpec((_RB, 1), lambda i: (i, 0)),
            pl.BlockSpec((_RB, 1), lambda i: (i, 0)),
            full((2 * (_NODE + _HID) + 1, _HID)),
            full((1, _HID)),
            full((_HID, 1)),
            full((1, 1)),
        ],
        out_specs=pl.BlockSpec((_GB, 1), lambda i: (i, 0)),
        out_shape=jax.ShapeDtypeStruct((_B, 1), jnp.float32),
    )(x, xpp, am, ax, wl1, bl1, wg, bgs)


def kernel(x, edge_index, edge_attr, action, W1, b1, W2, b2, Wl1, bl1, Wg, bg):
    src = edge_index[0]
    dst = edge_index[1]

    gs, gd = _sc_gather(x, src, dst)
    w1ap = jnp.kron(jnp.eye(8, dtype=jnp.float32), W1[:_NODE])
    w1bp = jnp.kron(jnp.eye(8, dtype=jnp.float32), W1[_NODE:2 * _NODE])
    w1cp = jnp.kron(jnp.eye(32, dtype=jnp.float32), W1[2 * _NODE:])
    w2p = jnp.kron(jnp.eye(4, dtype=jnp.float32), W2)
    ea32 = (edge_attr.T.reshape(_EATTR, _E // 32, 32)
            .transpose(1, 2, 0).reshape(_E // 32, 128))
    msgf = _edge_mlp(gs.reshape(_E // 8, 128), gd.reshape(_E // 8, 128),
                     ea32,
                     w1ap, w1bp, w1cp,
                     jnp.tile(b1, 4).reshape(1, 128), w2p,
                     jnp.tile(b2, 4).reshape(1, 128))
    xpp = _sc_scatter(src, msgf.reshape(_E, _HID))

    am = action[:, :_NNODES].reshape(_N, 1)
    ax = jnp.concatenate(
        [jnp.zeros((_B, _NNODES - _NFACT), jnp.float32), action[:, _NNODES:]],
        axis=1).reshape(_N, 1)

    out = _head(x, xpp, am, ax,
                Wl1, bl1.reshape(1, _HID), Wg, bg.reshape(1, 1))
    return out.reshape(_B)
